# baseline (speedup 1.0000x reference)
import functools

import jax
import jax.numpy as jnp
from jax import lax
from jax.experimental import pallas as pl
from jax.experimental.pallas import tpu as pltpu
from jax.experimental.pallas import tpu_sc as plsc

_D = 256
_B = 100000
_NC = 2
_NS = 16
_NW = _NC * _NS
_C = 80
_NBUF = 6
_K = 3
_STEPS = 40
_TAIL = 8
_BPW = (_STEPS - 1) * _C + _TAIL
_IPW = _STEPS * _C
_BPAD = _BPW * _NW
_LOOPED = (_STEPS - _K) // _NBUF * _NBUF

_mesh = plsc.VectorSubcoreMesh(core_axis_name="c", subcore_axis_name="s")


@functools.partial(
    pl.kernel,
    out_type=jax.ShapeDtypeStruct((_BPAD, _D), jnp.float32),
    mesh=_mesh,
    scratch_types=[
        pltpu.VMEM((_IPW,), jnp.int32),
        pltpu.VMEM((_NBUF, _C, _D), jnp.float32),
    ]
    + [pltpu.SemaphoreType.DMA] * (2 * _NBUF),
)
def _gather_rows(x_hbm, idx_hbm, out_hbm, idx_v, rows_v, *sems):
    gsems = sems[:_NBUF]
    wsems = sems[_NBUF:]
    wid = lax.axis_index("s") * _NC + lax.axis_index("c")
    base = wid * _BPW

    pltpu.sync_copy(idx_hbm.at[pl.ds(wid * _IPW, _IPW)], idx_v)

    def start_gather(slot, chunk):
        pltpu.async_copy(
            x_hbm.at[idx_v.at[pl.ds(chunk * _C, _C)]], rows_v.at[slot], gsems[slot]
        )

    def wait_gather(slot):
        pltpu.make_async_copy(
            x_hbm.at[idx_v.at[pl.ds(0, _C)]], rows_v.at[slot], gsems[slot]
        ).wait()

    def start_write(slot, chunk, rows=_C):
        pltpu.async_copy(
            rows_v.at[slot, pl.ds(0, rows)],
            out_hbm.at[pl.ds(base + chunk * _C, rows)],
            wsems[slot],
        )

    def wait_write(slot, rows=_C):
        pltpu.make_async_copy(
            rows_v.at[slot, pl.ds(0, rows)],
            out_hbm.at[pl.ds(base, rows)],
            wsems[slot],
        ).wait()

    for c in range(_K):
        start_gather(c, c)

    @pl.loop(0, _LOOPED, step=_NBUF)
    def _(j):
        for b in range(_NBUF):
            cj = j + b
            slot = b
            nslot = (b + _K) % _NBUF
            wait_gather(slot)
            start_write(slot, cj)

            @pl.when(cj >= _NBUF - _K)
            def _():
                wait_write(nslot)

            start_gather(nslot, cj + _K)

    for t in range(_LOOPED, _STEPS):
        slot = t % _NBUF
        wait_gather(slot)
        if t == _STEPS - 1:
            start_write(slot, t, _TAIL)
        else:
            start_write(slot, t)
        nslot = (t + _K) % _NBUF
        wait_write(nslot)
        if t + _K < _STEPS:
            start_gather(nslot, t + _K)

    for t in range(_STEPS - (_NBUF - _K), _STEPS):
        wait_write(t % _NBUF, _TAIL if t == _STEPS - 1 else _C)


def kernel(x, inds):
    idx = inds[:, 0].astype(jnp.int32)
    idx = jnp.concatenate([idx, jnp.zeros((_BPAD - _B,), jnp.int32)])
    idx = idx.reshape(_NW, _BPW)
    idx = jnp.pad(idx, ((0, 0), (0, _IPW - _BPW))).reshape(-1)
    out = _gather_rows(x, idx)
    return out[:_B]
